# Initial kernel scaffold; baseline (speedup 1.0000x reference)
import jax
import jax.numpy as jnp
from jax.experimental import pallas as pl


def kernel(X, vertex_idx, edge_idx):
    raise NotImplementedError("write your pallas kernel here")


# speedup vs baseline: 6.0609x; 6.0609x over previous
import functools

import jax
import jax.numpy as jnp
from jax import lax
from jax.experimental import pallas as pl
from jax.experimental.pallas import tpu as pltpu
from jax.experimental.pallas import tpu_sc as plsc

N_NODES = 10000
N_EDGES = 2500
N_PAIRS = 320000
D = 128

NC = 2
NS = 16
CH = 80
NW = NC * NS
RPW = N_PAIRS // NW // CH
RPT = 2 * RPW
BLK = 25
NBLK = RPW // BLK

NPAD = 10240
EPAD = 2560
NV_T = NPAD // NS
NE_T = EPAD // NS

_mesh = plsc.VectorSubcoreMesh(
    core_axis_name="c", subcore_axis_name="s", num_cores=NC, num_subcores=NS)


def _deg_side(idx_hbm, deg_sh, inv_out, span, idx_v, ones_v, buf_v, s):
    pltpu.sync_copy(buf_v.at[pl.ds(0, span)], deg_sh.at[pl.ds(s * span, span)])
    pltpu.sync_copy(idx_hbm.at[2 * s], idx_v.at[pl.ds(0, RPW)])
    pltpu.sync_copy(idx_hbm.at[2 * s + 1], idx_v.at[pl.ds(RPW, RPW)])
    plsc.subcore_barrier()

    def chunk(j, carry):
        pltpu.sync_copy(ones_v, deg_sh.at[idx_v.at[j]], add=True)
        return carry

    lax.fori_loop(0, RPT, chunk, 0)
    plsc.subcore_barrier()
    pltpu.sync_copy(deg_sh.at[pl.ds(s * span, span)], buf_v.at[pl.ds(0, span)])
    for k in range(span // 16):
        d = buf_v[pl.ds(k * 16, 16)]
        buf_v[pl.ds(k * 16, 16)] = jnp.where(
            d > 0.0, 1.0 / jnp.maximum(d, 1.0), 0.0)
    pltpu.sync_copy(buf_v.at[pl.ds(0, span)], inv_out.at[pl.ds(s * span, span)])


@functools.partial(
    pl.kernel,
    out_type=(jax.ShapeDtypeStruct((NPAD,), jnp.float32),
              jax.ShapeDtypeStruct((EPAD,), jnp.float32)),
    mesh=_mesh,
    compiler_params=pltpu.CompilerParams(use_tc_tiling_on_sc=False),
    scratch_types=[
        pltpu.VMEM_SHARED((NPAD,), jnp.float32),
        pltpu.VMEM_SHARED((EPAD,), jnp.float32),
        pltpu.VMEM((RPT, CH), jnp.int32),
        pltpu.VMEM((CH,), jnp.float32),
        pltpu.VMEM((NV_T,), jnp.float32),
    ],
)
def _deg_kernel(vidx_hbm, eidx_hbm, invv_out, inve_out,
                degv_sh, dege_sh, idx_v, ones_v, buf_v):
    c = lax.axis_index("c")
    s = lax.axis_index("s")
    for k in range(CH // 16):
        ones_v[pl.ds(k * 16, 16)] = jnp.ones((16,), jnp.float32)
    for k in range(NV_T // 16):
        buf_v[pl.ds(k * 16, 16)] = jnp.zeros((16,), jnp.float32)

    @pl.when(c == 0)
    def _():
        _deg_side(vidx_hbm, degv_sh, invv_out, NV_T, idx_v, ones_v, buf_v, s)

    @pl.when(c == 1)
    def _():
        _deg_side(eidx_hbm, dege_sh, inve_out, NE_T, idx_v, ones_v, buf_v, s)


@functools.partial(
    pl.kernel,
    out_type=jax.ShapeDtypeStruct((NC, EPAD, D), jnp.float32),
    mesh=_mesh,
    compiler_params=pltpu.CompilerParams(use_tc_tiling_on_sc=False),
    scratch_types=[
        pltpu.VMEM_SHARED((NPAD, D), jnp.float32),
        pltpu.VMEM_SHARED((EPAD, D), jnp.float32),
        pltpu.VMEM((BLK, CH), jnp.int32),
        pltpu.VMEM((BLK, CH), jnp.int32),
        pltpu.VMEM((CH, D), jnp.float32),
        pltpu.SemaphoreType.DMA,
    ],
)
def _v2e_kernel(xn_hbm, vidx_hbm, eidx_hbm, zeros_hbm, parts_out,
                tab_sh, acc_sh, vidx_v, eidx_v, rows_v, sem):
    c = lax.axis_index("c")
    s = lax.axis_index("s")
    w = c * NS + s
    pltpu.sync_copy(xn_hbm.at[pl.ds(s * NV_T, NV_T)],
                    tab_sh.at[pl.ds(s * NV_T, NV_T)])
    pltpu.sync_copy(zeros_hbm.at[pl.ds(0, NE_T)],
                    acc_sh.at[pl.ds(s * NE_T, NE_T)])
    plsc.subcore_barrier()

    def block(b, carry):
        pltpu.sync_copy(vidx_hbm.at[w, pl.ds(b * BLK, BLK)], vidx_v)
        pltpu.sync_copy(eidx_hbm.at[w, pl.ds(b * BLK, BLK)], eidx_v)

        def chunk(j, carry2):
            pltpu.async_copy(tab_sh.at[vidx_v.at[j]], rows_v, sem).wait()
            pltpu.sync_copy(rows_v, acc_sh.at[eidx_v.at[j]], add=True)
            return carry2

        lax.fori_loop(0, BLK, chunk, 0)
        return carry

    lax.fori_loop(0, NBLK, block, 0)
    plsc.subcore_barrier()
    pltpu.sync_copy(acc_sh.at[pl.ds(s * NE_T, NE_T)],
                    parts_out.at[c, pl.ds(s * NE_T, NE_T)])


@functools.partial(
    pl.kernel,
    out_type=jax.ShapeDtypeStruct((NC, NPAD, D), jnp.float32),
    mesh=_mesh,
    compiler_params=pltpu.CompilerParams(use_tc_tiling_on_sc=False),
    scratch_types=[
        pltpu.VMEM_SHARED((EPAD, D), jnp.float32),
        pltpu.VMEM_SHARED((NPAD, D), jnp.float32),
        pltpu.VMEM((BLK, CH), jnp.int32),
        pltpu.VMEM((BLK, CH), jnp.int32),
        pltpu.VMEM((CH, D), jnp.float32),
        pltpu.SemaphoreType.DMA,
    ],
)
def _e2v_kernel(en_hbm, vidx_hbm, eidx_hbm, zeros_hbm, parts_out,
                tab_sh, acc_sh, vidx_v, eidx_v, rows_v, sem):
    c = lax.axis_index("c")
    s = lax.axis_index("s")
    w = c * NS + s
    pltpu.sync_copy(en_hbm.at[pl.ds(s * NE_T, NE_T)],
                    tab_sh.at[pl.ds(s * NE_T, NE_T)])
    pltpu.sync_copy(zeros_hbm.at[pl.ds(0, NV_T)],
                    acc_sh.at[pl.ds(s * NV_T, NV_T)])
    plsc.subcore_barrier()

    def block(b, carry):
        pltpu.sync_copy(vidx_hbm.at[w, pl.ds(b * BLK, BLK)], vidx_v)
        pltpu.sync_copy(eidx_hbm.at[w, pl.ds(b * BLK, BLK)], eidx_v)

        def chunk(j, carry2):
            pltpu.async_copy(tab_sh.at[eidx_v.at[j]], rows_v, sem).wait()
            pltpu.sync_copy(rows_v, acc_sh.at[vidx_v.at[j]], add=True)
            return carry2

        lax.fori_loop(0, BLK, chunk, 0)
        return carry

    lax.fori_loop(0, NBLK, block, 0)
    plsc.subcore_barrier()
    pltpu.sync_copy(acc_sh.at[pl.ds(s * NV_T, NV_T)],
                    parts_out.at[c, pl.ds(s * NV_T, NV_T)])


def _xnorm_body(x_ref, iv_ref, o_ref):
    o_ref[...] = x_ref[...] * iv_ref[...]


_xnorm = pl.pallas_call(
    _xnorm_body,
    grid=(10,),
    in_specs=[pl.BlockSpec((1024, D), lambda i: (i, 0)),
              pl.BlockSpec((1024, 1), lambda i: (i, 0))],
    out_specs=pl.BlockSpec((1024, D), lambda i: (i, 0)),
    out_shape=jax.ShapeDtypeStruct((NPAD, D), jnp.float32),
)


def _ecomb_body(p_ref, iv_ref, ef_ref, en_ref):
    ef = p_ref[0] + p_ref[1]
    ef_ref[...] = ef
    en_ref[...] = ef * iv_ref[...]


_ecomb = pl.pallas_call(
    _ecomb_body,
    grid=(1,),
    in_specs=[pl.BlockSpec((2, N_EDGES, D), lambda i: (0, 0, 0)),
              pl.BlockSpec((N_EDGES, 1), lambda i: (0, 0))],
    out_specs=[pl.BlockSpec((N_EDGES, D), lambda i: (0, 0)),
               pl.BlockSpec((N_EDGES, D), lambda i: (0, 0))],
    out_shape=[jax.ShapeDtypeStruct((N_EDGES, D), jnp.float32),
               jax.ShapeDtypeStruct((N_EDGES, D), jnp.float32)],
)


def _ncomb_body(p_ref, o_ref):
    o_ref[...] = p_ref[0] + p_ref[1]


_ncomb = pl.pallas_call(
    _ncomb_body,
    grid=(10,),
    in_specs=[pl.BlockSpec((2, 1000, D), lambda i: (0, i, 0))],
    out_specs=pl.BlockSpec((1000, D), lambda i: (i, 0)),
    out_shape=jax.ShapeDtypeStruct((N_NODES, D), jnp.float32),
)


def kernel(X, vertex_idx, edge_idx):
    vi = vertex_idx.astype(jnp.int32).reshape(NW, RPW, CH)
    ei = edge_idx.astype(jnp.int32).reshape(NW, RPW, CH)
    zeros = jnp.zeros((NV_T, D), jnp.float32)

    invv_p, inve_p = _deg_kernel(vi, ei)
    inve = inve_p[:N_EDGES].reshape(N_EDGES, 1)

    x_pad = jnp.concatenate(
        [X.astype(jnp.float32), jnp.zeros((NPAD - N_NODES, D), jnp.float32)],
        axis=0)
    xn = _xnorm(x_pad, invv_p.reshape(NPAD, 1))
    eparts = _v2e_kernel(xn, vi, ei, zeros)
    edge_feat, edge_norm = _ecomb(eparts[:, :N_EDGES], inve)

    en_pad = jnp.concatenate(
        [edge_norm, jnp.zeros((EPAD - N_EDGES, D), jnp.float32)], axis=0)
    nparts = _e2v_kernel(en_pad, vi, ei, zeros)
    node_feat = _ncomb(nparts[:, :N_NODES])
    return node_feat, edge_feat


# speedup vs baseline: 9.9161x; 1.6361x over previous
import functools

import jax
import jax.numpy as jnp
from jax import lax
from jax.experimental import pallas as pl
from jax.experimental.pallas import tpu as pltpu
from jax.experimental.pallas import tpu_sc as plsc

N_NODES = 10000
N_EDGES = 2500
N_PAIRS = 320000
D = 128

NC = 2
NS = 16
CH = 80
NW = NC * NS
RPW = N_PAIRS // NW // CH
RPT = 2 * RPW
BLK = 25
NBLK = RPW // BLK
PPW = N_PAIRS // NW
IBLK = PPW // NBLK
GCH = 400
GPB = IBLK // GCH
ECH = 160
EPB = 12
ETAIL = IBLK - EPB * ECH

NPAD = 10240
EPAD = 2560
NV_T = NPAD // NS
NE_T = EPAD // NS

_mesh = plsc.VectorSubcoreMesh(
    core_axis_name="c", subcore_axis_name="s", num_cores=NC, num_subcores=NS)


PPT = N_PAIRS // NS


def _deg_side(idx_hbm, deg_sh, inv_out, span, idx_v, ones_v, buf_v, s):
    pltpu.sync_copy(buf_v.at[pl.ds(0, span)], deg_sh.at[pl.ds(s * span, span)])
    pltpu.sync_copy(idx_hbm.at[pl.ds(s * PPT, PPT)], idx_v)
    plsc.subcore_barrier()
    pltpu.sync_copy(ones_v, deg_sh.at[idx_v], add=True)
    plsc.subcore_barrier()
    pltpu.sync_copy(deg_sh.at[pl.ds(s * span, span)], buf_v.at[pl.ds(0, span)])
    for k in range(span // 16):
        d = buf_v[pl.ds(k * 16, 16)]
        buf_v[pl.ds(k * 16, 16)] = jnp.where(
            d > 0.0, 1.0 / jnp.maximum(d, 1.0), 0.0)
    pltpu.sync_copy(buf_v.at[pl.ds(0, span)], inv_out.at[pl.ds(s * span, span)])


@functools.partial(
    pl.kernel,
    out_type=(jax.ShapeDtypeStruct((NPAD,), jnp.float32),
              jax.ShapeDtypeStruct((EPAD,), jnp.float32)),
    mesh=_mesh,
    compiler_params=pltpu.CompilerParams(use_tc_tiling_on_sc=False),
    scratch_types=[
        pltpu.VMEM_SHARED((NPAD,), jnp.float32),
        pltpu.VMEM_SHARED((EPAD,), jnp.float32),
        pltpu.VMEM((PPT,), jnp.int32),
        pltpu.VMEM((PPT,), jnp.float32),
        pltpu.VMEM((NV_T,), jnp.float32),
    ],
)
def _deg_kernel(vidx_hbm, eidx_hbm, ones_hbm, invv_out, inve_out,
                degv_sh, dege_sh, idx_v, ones_v, buf_v):
    c = lax.axis_index("c")
    s = lax.axis_index("s")
    pltpu.sync_copy(ones_hbm, ones_v)
    for k in range(NV_T // 16):
        buf_v[pl.ds(k * 16, 16)] = jnp.zeros((16,), jnp.float32)

    @pl.when(c == 0)
    def _():
        _deg_side(vidx_hbm, degv_sh, invv_out, NV_T, idx_v, ones_v, buf_v, s)

    @pl.when(c == 1)
    def _():
        _deg_side(eidx_hbm, dege_sh, inve_out, NE_T, idx_v, ones_v, buf_v, s)


@functools.partial(
    pl.kernel,
    out_type=jax.ShapeDtypeStruct((NC, EPAD, D), jnp.float32),
    mesh=_mesh,
    compiler_params=pltpu.CompilerParams(use_tc_tiling_on_sc=False),
    scratch_types=[
        pltpu.VMEM_SHARED((EPAD, D), jnp.float32),
        pltpu.VMEM((IBLK,), jnp.int32),
        pltpu.VMEM((IBLK,), jnp.int32),
        pltpu.VMEM((GCH, D), jnp.float32),
        pltpu.VMEM((GCH, D), jnp.float32),
        pltpu.SemaphoreType.DMA,
        pltpu.SemaphoreType.DMA,
    ],
)
def _v2e_kernel(xn_hbm, vidx_hbm, eidx_hbm, zeros_hbm, parts_out,
                acc_sh, vidx_v, eidx_v, rows_a, rows_b, sem, ssem):
    c = lax.axis_index("c")
    s = lax.axis_index("s")
    w = c * NS + s
    pltpu.sync_copy(zeros_hbm.at[pl.ds(0, NE_T)],
                    acc_sh.at[pl.ds(s * NE_T, NE_T)])
    plsc.subcore_barrier()

    def block(b, carry):
        pltpu.sync_copy(vidx_hbm.at[w, pl.ds(b * IBLK, IBLK)], vidx_v)
        pltpu.sync_copy(eidx_hbm.at[w, pl.ds(b * IBLK, IBLK)], eidx_v)
        pltpu.async_copy(xn_hbm.at[vidx_v.at[pl.ds(0, GCH)]], rows_a, sem)

        def step(j, carry2):
            @pl.when(j % 2 == 0)
            def _():
                pltpu.make_async_copy(
                    xn_hbm.at[vidx_v.at[pl.ds(j * GCH, GCH)]],
                    rows_a, sem).wait()

                @pl.when(j > 0)
                def _():
                    pltpu.make_async_copy(
                        rows_b, acc_sh.at[pl.ds(0, GCH)], ssem).wait()

                @pl.when(j < GPB - 1)
                def _():
                    pltpu.async_copy(
                        xn_hbm.at[vidx_v.at[pl.ds((j + 1) * GCH, GCH)]],
                        rows_b, sem)

                pltpu.async_copy(
                    rows_a, acc_sh.at[eidx_v.at[pl.ds(j * GCH, GCH)]],
                    ssem, add=True)

            @pl.when(j % 2 == 1)
            def _():
                pltpu.make_async_copy(
                    xn_hbm.at[vidx_v.at[pl.ds(j * GCH, GCH)]],
                    rows_b, sem).wait()
                pltpu.make_async_copy(
                    rows_a, acc_sh.at[pl.ds(0, GCH)], ssem).wait()

                @pl.when(j < GPB - 1)
                def _():
                    pltpu.async_copy(
                        xn_hbm.at[vidx_v.at[pl.ds((j + 1) * GCH, GCH)]],
                        rows_a, sem)

                pltpu.async_copy(
                    rows_b, acc_sh.at[eidx_v.at[pl.ds(j * GCH, GCH)]],
                    ssem, add=True)

            return carry2

        lax.fori_loop(0, GPB, step, 0)
        pltpu.make_async_copy(rows_a, acc_sh.at[pl.ds(0, GCH)], ssem).wait()
        return carry

    lax.fori_loop(0, NBLK, block, 0)
    plsc.subcore_barrier()
    pltpu.sync_copy(acc_sh.at[pl.ds(s * NE_T, NE_T)],
                    parts_out.at[c, pl.ds(s * NE_T, NE_T)])


@functools.partial(
    pl.kernel,
    out_type=jax.ShapeDtypeStruct((NC, NPAD, D), jnp.float32),
    mesh=_mesh,
    compiler_params=pltpu.CompilerParams(use_tc_tiling_on_sc=False),
    scratch_types=[
        pltpu.VMEM_SHARED((NPAD, D), jnp.float32),
        pltpu.VMEM((IBLK,), jnp.int32),
        pltpu.VMEM((IBLK,), jnp.int32),
        pltpu.VMEM((ECH, D), jnp.float32),
        pltpu.VMEM((ECH, D), jnp.float32),
        pltpu.SemaphoreType.DMA,
        pltpu.SemaphoreType.DMA,
    ],
)
def _e2v_kernel(en_hbm, vidx_hbm, eidx_hbm, zeros_hbm, parts_out,
                acc_sh, vidx_v, eidx_v, rows_a, rows_b, sem, ssem):
    c = lax.axis_index("c")
    s = lax.axis_index("s")
    w = c * NS + s
    pltpu.sync_copy(zeros_hbm.at[pl.ds(0, NV_T)],
                    acc_sh.at[pl.ds(s * NV_T, NV_T)])
    plsc.subcore_barrier()

    def block(b, carry):
        pltpu.sync_copy(vidx_hbm.at[w, pl.ds(b * IBLK, IBLK)], vidx_v)
        pltpu.sync_copy(eidx_hbm.at[w, pl.ds(b * IBLK, IBLK)], eidx_v)
        pltpu.async_copy(en_hbm.at[eidx_v.at[pl.ds(0, ECH)]], rows_a, sem)

        def step(j, carry2):
            @pl.when(j % 2 == 0)
            def _():
                pltpu.make_async_copy(
                    en_hbm.at[eidx_v.at[pl.ds(j * ECH, ECH)]],
                    rows_a, sem).wait()

                @pl.when(j > 0)
                def _():
                    pltpu.make_async_copy(
                        rows_b, acc_sh.at[pl.ds(0, ECH)], ssem).wait()

                @pl.when(j < EPB - 1)
                def _():
                    pltpu.async_copy(
                        en_hbm.at[eidx_v.at[pl.ds((j + 1) * ECH, ECH)]],
                        rows_b, sem)

                pltpu.async_copy(
                    rows_a, acc_sh.at[vidx_v.at[pl.ds(j * ECH, ECH)]],
                    ssem, add=True)

            @pl.when(j % 2 == 1)
            def _():
                pltpu.make_async_copy(
                    en_hbm.at[eidx_v.at[pl.ds(j * ECH, ECH)]],
                    rows_b, sem).wait()
                pltpu.make_async_copy(
                    rows_a, acc_sh.at[pl.ds(0, ECH)], ssem).wait()

                @pl.when(j < EPB - 1)
                def _():
                    pltpu.async_copy(
                        en_hbm.at[eidx_v.at[pl.ds((j + 1) * ECH, ECH)]],
                        rows_a, sem)

                @pl.when(j == EPB - 1)
                def _():
                    pltpu.async_copy(
                        en_hbm.at[eidx_v.at[pl.ds(EPB * ECH, ETAIL)]],
                        rows_a.at[pl.ds(0, ETAIL)], sem)

                pltpu.async_copy(
                    rows_b, acc_sh.at[vidx_v.at[pl.ds(j * ECH, ECH)]],
                    ssem, add=True)

            return carry2

        lax.fori_loop(0, EPB, step, 0)
        pltpu.make_async_copy(
            en_hbm.at[eidx_v.at[pl.ds(EPB * ECH, ETAIL)]],
            rows_a.at[pl.ds(0, ETAIL)], sem).wait()
        pltpu.make_async_copy(rows_b, acc_sh.at[pl.ds(0, ECH)], ssem).wait()
        pltpu.async_copy(
            rows_a.at[pl.ds(0, ETAIL)],
            acc_sh.at[vidx_v.at[pl.ds(EPB * ECH, ETAIL)]], ssem, add=True)
        pltpu.make_async_copy(
            rows_a.at[pl.ds(0, ETAIL)], acc_sh.at[pl.ds(0, ETAIL)],
            ssem).wait()
        return carry

    lax.fori_loop(0, NBLK, block, 0)
    plsc.subcore_barrier()
    pltpu.sync_copy(acc_sh.at[pl.ds(s * NV_T, NV_T)],
                    parts_out.at[c, pl.ds(s * NV_T, NV_T)])


def _xnorm_body(x_ref, iv_ref, o_ref):
    o_ref[...] = x_ref[...] * iv_ref[...]


_xnorm = pl.pallas_call(
    _xnorm_body,
    grid=(10,),
    in_specs=[pl.BlockSpec((1000, D), lambda i: (i, 0)),
              pl.BlockSpec((1000, 1), lambda i: (i, 0))],
    out_specs=pl.BlockSpec((1000, D), lambda i: (i, 0)),
    out_shape=jax.ShapeDtypeStruct((NPAD, D), jnp.float32),
)


def _ecomb_body(p_ref, iv_ref, ef_ref, en_ref):
    ef = p_ref[0] + p_ref[1]
    ef_ref[...] = ef[:N_EDGES]
    en_ref[...] = ef * iv_ref[...]


_ecomb = pl.pallas_call(
    _ecomb_body,
    grid=(1,),
    in_specs=[pl.BlockSpec((2, EPAD, D), lambda i: (0, 0, 0)),
              pl.BlockSpec((EPAD, 1), lambda i: (0, 0))],
    out_specs=[pl.BlockSpec((N_EDGES, D), lambda i: (0, 0)),
               pl.BlockSpec((EPAD, D), lambda i: (0, 0))],
    out_shape=[jax.ShapeDtypeStruct((N_EDGES, D), jnp.float32),
               jax.ShapeDtypeStruct((EPAD, D), jnp.float32)],
)


def _ncomb_body(p_ref, o_ref):
    o_ref[...] = p_ref[0] + p_ref[1]


_ncomb = pl.pallas_call(
    _ncomb_body,
    grid=(10,),
    in_specs=[pl.BlockSpec((2, 1000, D), lambda i: (0, i, 0))],
    out_specs=pl.BlockSpec((1000, D), lambda i: (i, 0)),
    out_shape=jax.ShapeDtypeStruct((N_NODES, D), jnp.float32),
)


def kernel(X, vertex_idx, edge_idx):
    vi1 = vertex_idx.astype(jnp.int32)
    ei1 = edge_idx.astype(jnp.int32)
    vi2 = vi1.reshape(NW, PPW)
    ei2 = ei1.reshape(NW, PPW)
    zeros = jnp.zeros((NV_T, D), jnp.float32)
    ones = jnp.ones((PPT,), jnp.float32)

    invv_p, inve_p = _deg_kernel(vi1, ei1, ones)

    xn = _xnorm(X.astype(jnp.float32), invv_p[:N_NODES].reshape(N_NODES, 1))
    eparts = _v2e_kernel(xn, vi2, ei2, zeros)
    edge_feat, en_pad = _ecomb(eparts, inve_p.reshape(EPAD, 1))

    nparts = _e2v_kernel(en_pad, vi2, ei2, zeros)
    node_feat = _ncomb(nparts)
    return node_feat, edge_feat
